# baseline (speedup 1.0000x reference)
import jax
import jax.numpy as jnp
from jax.experimental import pallas as pl

_B, _SUP, _C, _V, _L = 4, 3, 32, 2048, 12
_K_LIST = [10, 20, 40]


def _knn_mask(a, k):
    vals, idx = jax.lax.top_k(a, k)
    full = jnp.full(a.shape, -1e20, a.dtype)
    bi = jnp.arange(a.shape[0])[:, None, None]
    ri = jnp.arange(a.shape[1])[None, :, None]
    return full.at[bi, ri, idx].set(vals)


def _combine_kernel(ys_ref, u_ref, beta_ref, o_ref):
    o_ref[0] = (
        jnp.dot(u_ref[...], ys_ref[0], preferred_element_type=jnp.float32)
        + beta_ref[...][:, :1]
    )


def kernel(x, support, W0, b0, W1, b1, W2, b2, W3, b3, Wf, bf):
    C = _C
    Ws = [W1, W2, W3]
    bs = [b1, b2, b3]
    Us = [Wf[:, 0:C] @ W0]
    for i in range(_SUP):
        WfB = Wf[:, C * (i + 1):C * (i + 2)]
        for t in range(4):
            Us.append(WfB @ Ws[i][:, C * t:C * (t + 1)])
    beta = bf + Wf[:, 0:C] @ b0
    for i in range(_SUP):
        beta = beta + Wf[:, C * (i + 1):C * (i + 2)] @ bs[i]
    U = jnp.concatenate(Us, axis=1)

    ys = [x]
    for i in range(_SUP):
        a = support[:, i]
        for k in _K_LIST:
            a1 = jax.nn.softmax(_knn_mask(a, k), axis=-1)
            ys.append(jnp.einsum('ncvl,nvw->ncwl', x, a1))
        a1 = jax.nn.softmax(a, axis=-1)
        ys.append(jnp.einsum('ncvl,nvw->ncwl', x, a1))
    ysc = jnp.concatenate(ys, axis=1)
    ysf = ysc.reshape(_B, 13 * C, _V * _L)

    VCH = 4
    out = pl.pallas_call(
        _combine_kernel,
        grid=(_B, VCH),
        in_specs=[
            pl.BlockSpec((1, 13 * C, _V * _L // VCH), lambda n, v: (n, 0, v)),
            pl.BlockSpec((C, 13 * C), lambda n, v: (0, 0)),
            pl.BlockSpec((C, 1), lambda n, v: (0, 0)),
        ],
        out_specs=pl.BlockSpec((1, C, _V * _L // VCH), lambda n, v: (n, 0, v)),
        out_shape=jax.ShapeDtypeStruct((_B, C, _V * _L), jnp.float32),
    )(ysf, U, beta[:, None])
    return out.reshape(_B, C, _V, _L)


# speedup vs baseline: 2.4258x; 2.4258x over previous
import functools

import jax
import jax.numpy as jnp
from jax.experimental import pallas as pl
from jax.experimental.pallas import tpu as pltpu

_B, _SUP, _C, _V, _L = 4, 3, 32, 2048, 12
_K_LIST = (10, 20, 40)
_R = 256
_VSTEPS = _V // _R


def _fused_kernel(a_ref, x_ref, u_ref, u0_ref, beta_ref, o_ref, z_ref):
    i = pl.program_id(1)
    v = pl.program_id(2)

    @pl.when(jnp.logical_and(i == 0, v == 0))
    def _init():
        xf = x_ref[0]
        z0 = jnp.dot(u0_ref[...], xf, preferred_element_type=jnp.float32)
        z0 = z0.reshape(_C, _L, _V) + beta_ref[...][:, :, None]
        o_ref[0] = z0.reshape(_C * _L, _V)

    @pl.when(v == 0)
    def _premix():
        xf = x_ref[0]
        for t in range(4):
            z_ref[t] = jnp.dot(u_ref[0, t], xf,
                               preferred_element_type=jnp.float32).reshape(
                                   _C, _L, _V)

    a = a_ref[0, 0]
    m = jnp.max(a, axis=1, keepdims=True)
    e = jnp.exp(a - m)

    cur = a
    ts = []
    for j in range(_K_LIST[-1]):
        mx = jnp.max(cur, axis=1, keepdims=True)
        if (j + 1) in _K_LIST:
            ts.append(mx)
        if j + 1 < _K_LIST[-1]:
            cur = jnp.where(cur >= mx, -3.0e38, cur)

    acc = o_ref[0]
    for t in range(4):
        if t < 3:
            g = jnp.where(a >= ts[t], e, 0.0)
        else:
            g = e
        s = jnp.sum(g, axis=1, keepdims=True)
        zt = z_ref[t, :, :, pl.ds(v * _R, _R)].reshape(_C * _L, _R)
        zs = zt * (1.0 / s).reshape(1, _R)
        acc = acc + jnp.dot(zs, g, preferred_element_type=jnp.float32)
    o_ref[0] = acc


def kernel(x, support, W0, b0, W1, b1, W2, b2, W3, b3, Wf, bf):
    C = _C
    Ws = [W1, W2, W3]
    bs = [b1, b2, b3]
    U0 = Wf[:, 0:C] @ W0
    Ust = jnp.stack([
        jnp.stack([Wf[:, C * (i + 1):C * (i + 2)] @ Ws[i][:, C * t:C * (t + 1)]
                   for t in range(4)])
        for i in range(_SUP)
    ])
    beta = bf + Wf[:, 0:C] @ b0
    for i in range(_SUP):
        beta = beta + Wf[:, C * (i + 1):C * (i + 2)] @ bs[i]

    xt = x.transpose(0, 1, 3, 2).reshape(_B, C, _L * _V)

    out = pl.pallas_call(
        _fused_kernel,
        grid=(_B, _SUP, _VSTEPS),
        in_specs=[
            pl.BlockSpec((1, 1, _R, _V), lambda n, i, v: (n, i, v, 0)),
            pl.BlockSpec((1, C, _L * _V), lambda n, i, v: (n, 0, 0)),
            pl.BlockSpec((1, 4, C, C), lambda n, i, v: (i, 0, 0, 0)),
            pl.BlockSpec((C, C), lambda n, i, v: (0, 0)),
            pl.BlockSpec((C, _L), lambda n, i, v: (0, 0)),
        ],
        out_specs=pl.BlockSpec((1, C * _L, _V), lambda n, i, v: (n, 0, 0)),
        out_shape=jax.ShapeDtypeStruct((_B, C * _L, _V), jnp.float32),
        scratch_shapes=[pltpu.VMEM((4, C, _L, _V), jnp.float32)],
    )(support, xt, Ust, U0, jnp.broadcast_to(beta[:, None], (C, _L)))
    return out.reshape(_B, C, _L, _V).transpose(0, 1, 3, 2)
